# baseline (speedup 1.0000x reference)
import functools
import math

import jax
import jax.numpy as jnp
from jax import lax
from jax.experimental import pallas as pl
from jax.experimental.pallas import tpu as pltpu
from jax.experimental.pallas import tpu_sc as plsc

N_NODES = 10000
N_EDGES = 320000
D_IN = 128
D_H = 64
AVG_NUM_NEIGHBOURS = 32.0
SCALE = 0.85
SHIFT = 0.12

LANES = 128
ROWS = N_EDGES // LANES
EDGE_BLOCK = 12800
MLP_GRID = N_EDGES // EDGE_BLOCK

N_PAD = 10240
NUM_CORES = 2
NUM_SUBCORES = 16
NTILES = NUM_CORES * NUM_SUBCORES
ROWS_PAD = 3200
SLICE = N_PAD // NUM_SUBCORES


ROW_BLOCK = EDGE_BLOCK // LANES
PAD_GRID = ROWS_PAD // ROW_BLOCK
BLOCKS_PER_TILE = PAD_GRID // NTILES


def _mlp_body(x_ref, w0_ref, w1_ref, w2_ref, o_ref):
    i = pl.program_id(0)

    @pl.when(i < MLP_GRID)
    def _():
        x = x_ref[...]
        h = jnp.dot(x, w0_ref[...], preferred_element_type=jnp.float32)
        h = h * lax.logistic(h)
        h = jnp.dot(h, w1_ref[...], preferred_element_type=jnp.float32)
        h = h * lax.logistic(h)
        z = h.T * w2_ref[...]
        o_ref[...] = jnp.sum(z, axis=0,
                             keepdims=True).reshape(1, ROW_BLOCK, LANES)

    @pl.when(i >= MLP_GRID)
    def _():
        o_ref[...] = jnp.zeros((1, ROW_BLOCK, LANES), jnp.float32)


def _mlp(edge_feats, w0, w1, w2s):
    return pl.pallas_call(
        _mlp_body,
        grid=(PAD_GRID,),
        in_specs=[
            pl.BlockSpec((EDGE_BLOCK, D_IN),
                         lambda i: (jnp.minimum(i, MLP_GRID - 1), 0)),
            pl.BlockSpec((D_IN, D_H), lambda i: (0, 0)),
            pl.BlockSpec((D_H, D_H), lambda i: (0, 0)),
            pl.BlockSpec((D_H, 1), lambda i: (0, 0)),
        ],
        out_specs=pl.BlockSpec((1, ROW_BLOCK, LANES), lambda i: (i, 0, 0)),
        out_shape=jax.ShapeDtypeStruct((PAD_GRID, ROW_BLOCK, LANES),
                                       jnp.float32),
    )(edge_feats, w0, w1, w2s)


def _scatter_body(idx_hbm, val_hbm, out_hbm, idx_v, val_v, buf_v, acc_sh):
    cid = lax.axis_index("c")
    sid = lax.axis_index("s")
    w = cid * NUM_SUBCORES + sid

    def zero_body(i, carry):
        buf_v[pl.ds(i * 16, 16)] = jnp.zeros((16,), jnp.float32)
        return carry

    lax.fori_loop(0, SLICE // 16, zero_body, 0)
    pltpu.sync_copy(buf_v, acc_sh.at[pl.ds(sid * SLICE, SLICE)])
    plsc.subcore_barrier()

    b0 = w * BLOCKS_PER_TILE
    pltpu.sync_copy(idx_hbm.at[pl.ds(b0, BLOCKS_PER_TILE)], idx_v)
    pltpu.sync_copy(val_hbm.at[pl.ds(b0, BLOCKS_PER_TILE)], val_v)

    def scat_blk(b, carry):
        def scat_row(j, carry2):
            pltpu.sync_copy(val_v.at[b, j], acc_sh.at[idx_v.at[b, j]],
                            add=True)
            return carry2

        return lax.fori_loop(0, ROW_BLOCK, scat_row, carry)

    lax.fori_loop(0, BLOCKS_PER_TILE, scat_blk, 0)

    plsc.subcore_barrier()

    pltpu.sync_copy(acc_sh.at[pl.ds(sid * SLICE, SLICE)], buf_v)
    pltpu.sync_copy(buf_v, out_hbm.at[cid, sid])


@functools.cache
def _make_scatter():
    mesh = plsc.VectorSubcoreMesh(core_axis_name="c", subcore_axis_name="s")
    return pl.kernel(
        _scatter_body,
        out_type=jax.ShapeDtypeStruct((NUM_CORES, NUM_SUBCORES, SLICE),
                                      jnp.float32),
        mesh=mesh,
        scratch_types=[
            pltpu.VMEM((BLOCKS_PER_TILE, ROW_BLOCK, LANES), jnp.int32),
            pltpu.VMEM((BLOCKS_PER_TILE, ROW_BLOCK, LANES), jnp.float32),
            pltpu.VMEM((SLICE,), jnp.float32),
            pltpu.VMEM_SHARED((N_PAD,), jnp.float32),
        ],
    )


def _comb_body(p_ref, o_ref):
    o_ref[...] = p_ref[0:1, :] + p_ref[1:2, :] + SHIFT


def _combine(partials):
    return pl.pallas_call(
        _comb_body,
        out_shape=jax.ShapeDtypeStruct((1, N_PAD), jnp.float32),
    )(partials)


def kernel(edge_feats, edge_index, num_nodes, W0, W1, W2):
    del num_nodes
    c = SCALE / math.sqrt(AVG_NUM_NEIGHBOURS)
    w2s = (W2 * c).astype(jnp.float32)
    vals3d = _mlp(edge_feats, W0, W1, w2s)
    idx3d = jnp.pad(edge_index[0].reshape(ROWS, LANES),
                    ((0, ROWS_PAD - ROWS), (0, 0)))
    idx3d = idx3d.reshape(PAD_GRID, ROW_BLOCK, LANES)
    partials = _make_scatter()(idx3d, vals3d)
    node = _combine(partials.reshape(NUM_CORES, N_PAD))
    return node[0, :N_NODES].reshape(N_NODES, 1)


# speedup vs baseline: 1.6040x; 1.6040x over previous
import functools
import math

import numpy as _np

import jax
import jax.numpy as jnp
from jax import lax
from jax.experimental import pallas as pl
from jax.experimental.pallas import tpu as pltpu
from jax.experimental.pallas import tpu_sc as plsc

N_NODES = 10000
N_EDGES = 320000
D_IN = 128
D_H = 64
AVG_NUM_NEIGHBOURS = 32.0
SCALE = 0.85
SHIFT = 0.12

LANES = 128
ROWS = N_EDGES // LANES
EDGE_BLOCK = 12800
MLP_GRID = N_EDGES // EDGE_BLOCK

N_PAD = 10240
NUM_CORES = 2
NUM_SUBCORES = 16
NTILES = NUM_CORES * NUM_SUBCORES
ROWS_PAD = 3200
SLICE = N_PAD // NUM_SUBCORES


ROW_BLOCK = EDGE_BLOCK // LANES
PAD_GRID = ROWS_PAD // ROW_BLOCK
BLOCKS_PER_TILE = PAD_GRID // NTILES


def _mlp_body(x_ref, w0_ref, w1_ref, w2_ref, o_ref):
    i = pl.program_id(0)

    @pl.when(i < MLP_GRID)
    def _():
        x = x_ref[...]
        h = jnp.dot(x, w0_ref[...], preferred_element_type=jnp.float32)
        h = h * lax.logistic(h)
        h = jnp.dot(h, w1_ref[...], preferred_element_type=jnp.float32)
        h = h * lax.logistic(h)
        z = h.T * w2_ref[...]
        o_ref[...] = jnp.sum(z, axis=0,
                             keepdims=True).reshape(1, ROW_BLOCK, LANES)

    @pl.when(i >= MLP_GRID)
    def _():
        o_ref[...] = jnp.zeros((1, ROW_BLOCK, LANES), jnp.float32)


def _mlp(edge_feats, w0, w1, w2s):
    return pl.pallas_call(
        _mlp_body,
        grid=(PAD_GRID,),
        in_specs=[
            pl.BlockSpec((EDGE_BLOCK, D_IN),
                         lambda i: (jnp.minimum(i, MLP_GRID - 1), 0)),
            pl.BlockSpec((D_IN, D_H), lambda i: (0, 0)),
            pl.BlockSpec((D_H, D_H), lambda i: (0, 0)),
            pl.BlockSpec((D_H, 1), lambda i: (0, 0)),
        ],
        out_specs=pl.BlockSpec((1, ROW_BLOCK, LANES), lambda i: (i, 0, 0)),
        out_shape=jax.ShapeDtypeStruct((PAD_GRID, ROW_BLOCK, LANES),
                                       jnp.float32),
    )(edge_feats, w0, w1, w2s)


def _scatter_body(idx_hbm, val_hbm, out_hbm, idx_v, val_v, buf_v, acc_sh):
    cid = lax.axis_index("c")
    sid = lax.axis_index("s")
    w = cid * NUM_SUBCORES + sid

    def zero_body(i, carry):
        buf_v[pl.ds(i * 16, 16)] = jnp.zeros((16,), jnp.float32)
        return carry

    lax.fori_loop(0, SLICE // 16, zero_body, 0)
    pltpu.sync_copy(buf_v, acc_sh.at[pl.ds(sid * SLICE, SLICE)])
    plsc.subcore_barrier()

    b0 = w * BLOCKS_PER_TILE
    pltpu.sync_copy(idx_hbm.at[pl.ds(b0, BLOCKS_PER_TILE)], idx_v)
    pltpu.sync_copy(val_hbm.at[pl.ds(b0, BLOCKS_PER_TILE)], val_v)

    def scat_blk(b, carry):
        def scat_row(j, carry2):
            pltpu.sync_copy(val_v.at[b, j], acc_sh.at[idx_v.at[b, j]],
                            add=True)
            return carry2

        return lax.fori_loop(0, ROW_BLOCK, scat_row, carry)

    lax.fori_loop(0, BLOCKS_PER_TILE, scat_blk, 0)

    plsc.subcore_barrier()

    pltpu.sync_copy(acc_sh.at[pl.ds(sid * SLICE, SLICE)], buf_v)
    pltpu.sync_copy(buf_v, out_hbm.at[cid, sid])


@functools.cache
def _make_scatter():
    mesh = plsc.VectorSubcoreMesh(core_axis_name="c", subcore_axis_name="s")
    return pl.kernel(
        _scatter_body,
        out_type=jax.ShapeDtypeStruct((NUM_CORES, NUM_SUBCORES, SLICE),
                                      jnp.float32),
        mesh=mesh,
        scratch_types=[
            pltpu.VMEM((BLOCKS_PER_TILE, ROW_BLOCK, LANES), jnp.int32),
            pltpu.VMEM((BLOCKS_PER_TILE, ROW_BLOCK, LANES), jnp.float32),
            pltpu.VMEM((SLICE,), jnp.float32),
            pltpu.VMEM_SHARED((N_PAD,), jnp.float32),
        ],
    )


def _comb_body(p_ref, o_ref):
    o_ref[...] = p_ref[0:1, :] + p_ref[1:2, :] + SHIFT


def _combine(partials):
    return pl.pallas_call(
        _comb_body,
        out_shape=jax.ShapeDtypeStruct((1, N_PAD), jnp.float32),
    )(partials)


def kernel(edge_feats, edge_index, num_nodes, W0, W1, W2):
    del num_nodes
    c = SCALE / math.sqrt(AVG_NUM_NEIGHBOURS)
    w2s = (W2 * c).astype(jnp.float32)
    vals3d = _mlp(edge_feats, W0, W1, w2s)
    pad_idx = (_np.arange((ROWS_PAD - ROWS) * LANES, dtype=_np.int32)
               % N_PAD).reshape(ROWS_PAD - ROWS, LANES)
    idx3d = jnp.concatenate(
        [edge_index[0].reshape(ROWS, LANES), jnp.asarray(pad_idx)], axis=0)
    idx3d = idx3d.reshape(PAD_GRID, ROW_BLOCK, LANES)
    partials = _make_scatter()(idx3d, vals3d)
    node = _combine(partials.reshape(NUM_CORES, N_PAD))
    return node[0, :N_NODES].reshape(N_NODES, 1)


# speedup vs baseline: 1.6530x; 1.0305x over previous
import functools
import math

import jax
import jax.numpy as jnp
from jax import lax
from jax.experimental import pallas as pl
from jax.experimental.pallas import tpu as pltpu
from jax.experimental.pallas import tpu_sc as plsc

N_NODES = 10000
N_EDGES = 320000
D_IN = 128
D_H = 64
AVG_NUM_NEIGHBOURS = 32.0
SCALE = 0.85
SHIFT = 0.12

LANES = 128
ROWS = N_EDGES // LANES
EDGE_BLOCK = 12800
MLP_GRID = N_EDGES // EDGE_BLOCK
ROW_BLOCK = EDGE_BLOCK // LANES

N_PAD = 10240
NUM_CORES = 2
NUM_SUBCORES = 16
SLICE = N_PAD // NUM_SUBCORES


def _mlp_body(x_ref, w0_ref, w1_ref, w2_ref, o_ref):
    x = x_ref[...]
    h = jnp.dot(x, w0_ref[...], preferred_element_type=jnp.float32)
    h = h * lax.logistic(h)
    h = jnp.dot(h, w1_ref[...], preferred_element_type=jnp.float32)
    h = h * lax.logistic(h)
    z = h.T * w2_ref[...]
    o_ref[...] = jnp.sum(z, axis=0, keepdims=True).reshape(1, ROW_BLOCK, LANES)


def _mlp(edge_feats, w0, w1, w2s):
    return pl.pallas_call(
        _mlp_body,
        grid=(MLP_GRID,),
        in_specs=[
            pl.BlockSpec((EDGE_BLOCK, D_IN), lambda i: (i, 0)),
            pl.BlockSpec((D_IN, D_H), lambda i: (0, 0)),
            pl.BlockSpec((D_H, D_H), lambda i: (0, 0)),
            pl.BlockSpec((D_H, 1), lambda i: (0, 0)),
        ],
        out_specs=pl.BlockSpec((1, ROW_BLOCK, LANES), lambda i: (i, 0, 0)),
        out_shape=jax.ShapeDtypeStruct((MLP_GRID, ROW_BLOCK, LANES),
                                       jnp.float32),
    )(edge_feats, w0, w1, w2s)


def _scatter_body(idx_hbm, val_hbm, out_hbm, idx_v, val_v, buf_v, acc_sh):
    cid = lax.axis_index("c")
    sid = lax.axis_index("s")
    w = sid * NUM_CORES + cid

    def zero_body(i, carry):
        buf_v[pl.ds(i * 16, 16)] = jnp.zeros((16,), jnp.float32)
        return carry

    lax.fori_loop(0, SLICE // 16, zero_body, 0)
    pltpu.sync_copy(buf_v, acc_sh.at[pl.ds(sid * SLICE, SLICE)])
    plsc.subcore_barrier()

    @pl.when(w < MLP_GRID)
    def _():
        pltpu.sync_copy(idx_hbm.at[pl.ds(w, 1)], idx_v)
        pltpu.sync_copy(val_hbm.at[pl.ds(w, 1)], val_v)

        def scat_row(j, carry):
            pltpu.sync_copy(val_v.at[0, j], acc_sh.at[idx_v.at[0, j]],
                            add=True)
            return carry

        lax.fori_loop(0, ROW_BLOCK, scat_row, 0)

    plsc.subcore_barrier()

    pltpu.sync_copy(acc_sh.at[pl.ds(sid * SLICE, SLICE)], buf_v)
    pltpu.sync_copy(buf_v, out_hbm.at[cid, sid])


@functools.cache
def _make_scatter():
    mesh = plsc.VectorSubcoreMesh(core_axis_name="c", subcore_axis_name="s")
    return pl.kernel(
        _scatter_body,
        out_type=jax.ShapeDtypeStruct((NUM_CORES, NUM_SUBCORES, SLICE),
                                      jnp.float32),
        mesh=mesh,
        scratch_types=[
            pltpu.VMEM((1, ROW_BLOCK, LANES), jnp.int32),
            pltpu.VMEM((1, ROW_BLOCK, LANES), jnp.float32),
            pltpu.VMEM((SLICE,), jnp.float32),
            pltpu.VMEM_SHARED((N_PAD,), jnp.float32),
        ],
    )


def _comb_body(p_ref, o_ref):
    o_ref[...] = p_ref[0:1, :] + p_ref[1:2, :] + SHIFT


def _combine(partials):
    return pl.pallas_call(
        _comb_body,
        out_shape=jax.ShapeDtypeStruct((1, N_PAD), jnp.float32),
    )(partials)


def kernel(edge_feats, edge_index, num_nodes, W0, W1, W2):
    del num_nodes
    c = SCALE / math.sqrt(AVG_NUM_NEIGHBOURS)
    w2s = (W2 * c).astype(jnp.float32)
    vals3d = _mlp(edge_feats, W0, W1, w2s)
    idx3d = edge_index[0].reshape(MLP_GRID, ROW_BLOCK, LANES)
    partials = _make_scatter()(idx3d, vals3d)
    node = _combine(partials.reshape(NUM_CORES, N_PAD))
    return node[0, :N_NODES].reshape(N_NODES, 1)
